# Initial kernel scaffold; baseline (speedup 1.0000x reference)
import jax
import jax.numpy as jnp
from jax.experimental import pallas as pl


def kernel(z, table):
    raise NotImplementedError("write your pallas kernel here")


# speedup vs baseline: 3.5525x; 3.5525x over previous
import functools

import jax
import jax.numpy as jnp
from jax import lax
from jax.experimental import pallas as pl
from jax.experimental.pallas import tpu as pltpu
from jax.experimental.pallas import tpu_sc as plsc

EMBED_DIM = 64
LN = 128
RPC = 4
NUM_WORKERS = 32


def _emb_body(z_rows, table, out3, idx_v, rows_v, sem):
    n_rows = z_rows.shape[0]
    per_w = n_rows // NUM_WORKERS
    n_chunks = per_w // RPC
    wid = lax.axis_index("s") * 2 + lax.axis_index("c")
    base = wid * per_w

    def chunk(i, carry):
        r0 = base + i * RPC
        pltpu.sync_copy(z_rows.at[pl.ds(r0, RPC)], idx_v)
        copies = [
            pltpu.async_copy(table.at[idx_v.at[j]], rows_v.at[j], sem)
            for j in range(RPC)
        ]
        for c in copies:
            c.wait()
        pltpu.sync_copy(rows_v, out3.at[pl.ds(r0, RPC)])
        return carry

    lax.fori_loop(0, n_chunks, chunk, 0)


@jax.jit
def kernel(z, table):
    B, S = z.shape
    n = B * S
    z_rows = z.reshape(n // LN, LN).astype(jnp.int32)
    table = table.at[0].set(jnp.zeros((table.shape[1],), table.dtype))

    mesh = plsc.VectorSubcoreMesh(core_axis_name="c", subcore_axis_name="s")
    out3 = pl.kernel(
        _emb_body,
        mesh=mesh,
        out_type=jax.ShapeDtypeStruct((n // LN, LN, EMBED_DIM), jnp.float32),
        scratch_types=[
            pltpu.VMEM((RPC, LN), jnp.int32),
            pltpu.VMEM((RPC, LN, EMBED_DIM), jnp.float32),
            pltpu.SemaphoreType.DMA,
        ],
        compiler_params=pltpu.CompilerParams(use_tc_tiling_on_sc=False),
    )(z_rows, table)
    return out3.reshape(B, S, EMBED_DIM)


# speedup vs baseline: 3.5650x; 1.0035x over previous
import functools

import jax
import jax.numpy as jnp
from jax import lax
from jax.experimental import pallas as pl
from jax.experimental.pallas import tpu as pltpu
from jax.experimental.pallas import tpu_sc as plsc

EMBED_DIM = 64
CHUNK = 512
NUM_WORKERS = 32


def _emb_body(z_rows, table, out3, idx_v, rows_v, sem_i, sem_g, sem_w):
    n_rows = z_rows.shape[0]
    per_w = n_rows // NUM_WORKERS
    wid = lax.axis_index("s") * 2 + lax.axis_index("c")
    base = wid * per_w

    pltpu.async_copy(z_rows.at[pl.ds(base, 1)], idx_v.at[0], sem_i.at[0])

    def pair(i, carry):
        for b in range(2):
            ci = 2 * i + b
            r0 = base + ci
            @pl.when(ci + 1 < per_w)
            def _():
                pltpu.async_copy(
                    z_rows.at[pl.ds(r0 + 1, 1)], idx_v.at[1 - b],
                    sem_i.at[1 - b])
            pltpu.make_async_copy(
                z_rows.at[pl.ds(r0, 1)], idx_v.at[b], sem_i.at[b]).wait()
            @pl.when(ci >= 2)
            def _():
                pltpu.make_async_copy(
                    rows_v.at[b], out3.at[pl.ds(r0, 1)], sem_w.at[b]).wait()
            pltpu.async_copy(
                table.at[idx_v.at[b, 0]], rows_v.at[b, 0], sem_g.at[b]).wait()
            pltpu.async_copy(rows_v.at[b], out3.at[pl.ds(r0, 1)], sem_w.at[b])
        return carry

    lax.fori_loop(0, per_w // 2, pair, 0)

    for b in range(2):
        r0 = base + per_w - 2 + b
        pltpu.make_async_copy(
            rows_v.at[b], out3.at[pl.ds(r0, 1)], sem_w.at[b]).wait()


@jax.jit
def kernel(z, table):
    B, S = z.shape
    n = B * S
    z_rows = z.reshape(n // CHUNK, CHUNK).astype(jnp.int32)
    table = table.at[0].set(jnp.zeros((table.shape[1],), table.dtype))

    mesh = plsc.VectorSubcoreMesh(core_axis_name="c", subcore_axis_name="s")
    out3 = pl.kernel(
        _emb_body,
        mesh=mesh,
        out_type=jax.ShapeDtypeStruct((n // CHUNK, CHUNK, EMBED_DIM),
                                      jnp.float32),
        scratch_types=[
            pltpu.VMEM((2, 1, CHUNK), jnp.int32),
            pltpu.VMEM((2, 1, CHUNK, EMBED_DIM), jnp.float32),
            pltpu.SemaphoreType.DMA((2,)),
            pltpu.SemaphoreType.DMA((2,)),
            pltpu.SemaphoreType.DMA((2,)),
        ],
        compiler_params=pltpu.CompilerParams(use_tc_tiling_on_sc=False),
    )(z_rows, table)
    return out3.reshape(B, S, EMBED_DIM)
